# baseline (speedup 1.0000x reference)
import dataclasses
import functools

import jax
import jax.numpy as jnp
from jax.experimental import pallas as pl
from jax.experimental.pallas import tpu as pltpu
from jax.experimental.pallas import tpu_sc as plsc

_INTERIM = 256
_M = 128
_C_IN = 2 * _INTERIM + 4
_C_OUT = 2 * _M + 4
_L = 16
_CHUNK = 32
_SCALE = 255.0 / 1.4


def _interp_block(x_vmem, q_vmem, o_vmem):
    @plsc.parallel_loop(0, _CHUNK, 1, unroll=2)
    def _row(r):
        row = jnp.full((_L,), r, dtype=jnp.int32)

        for g in range(0, _M, _L):
            x = q_vmem[r, pl.ds(g, _L)]
            t = x * _SCALE
            t = jnp.minimum(jnp.maximum(t, 0.0), float(_INTERIM - 1))
            lo = jnp.minimum(t.astype(jnp.int32), _INTERIM - 2)
            frac = t - lo.astype(jnp.float32)
            p_lo = plsc.load_gather(x_vmem, [row, lo])
            p_hi = plsc.load_gather(x_vmem, [row, lo + 1])
            t_lo = plsc.load_gather(x_vmem, [row, lo + _INTERIM])
            t_hi = plsc.load_gather(x_vmem, [row, lo + (_INTERIM + 1)])
            o_vmem[r, pl.ds(g, _L)] = p_lo + frac * (p_hi - p_lo)
            o_vmem[r, pl.ds(g + _M, _L)] = t_lo + frac * (t_hi - t_lo)

    iota = jax.lax.iota(jnp.int32, _L)

    @plsc.parallel_loop(0, _CHUNK, _L)
    def _pass(rb):
        rows = iota + rb
        for c in range(4):
            v = plsc.load_gather(
                x_vmem, [rows, jnp.full((_L,), 2 * _INTERIM + c, jnp.int32)])
            plsc.store_scatter(
                o_vmem, [rows, jnp.full((_L,), 2 * _M + c, jnp.int32)], v)


@functools.cache
def _build(batch):
    mesh = plsc.VectorSubcoreMesh(core_axis_name="c", subcore_axis_name="s")
    cp = pltpu.CompilerParams()
    if "needs_layout_passes" in pltpu.CompilerParams.__dataclass_fields__:
        cp = dataclasses.replace(cp, needs_layout_passes=False)
    if "use_tc_tiling_on_sc" in pltpu.CompilerParams.__dataclass_fields__:
        cp = dataclasses.replace(cp, use_tc_tiling_on_sc=False)

    @functools.partial(
        pl.kernel,
        out_type=jax.ShapeDtypeStruct((batch, _C_OUT), jnp.float32),
        mesh=mesh,
        compiler_params=cp,
    )
    def run(x_hbm, q_hbm, o_hbm):
        pltpu.emit_pipeline(
            _interp_block,
            grid=(batch // _CHUNK,),
            in_specs=[
                pl.BlockSpec((_CHUNK, _C_IN), lambda i: (i, 0)),
                pl.BlockSpec((_CHUNK, _M), lambda i: (i, 0)),
            ],
            out_specs=[pl.BlockSpec((_CHUNK, _C_OUT), lambda i: (i, 0))],
            core_axis_name=("c", "s"),
            dimension_semantics=(pltpu.PARALLEL,),
        )(x_hbm, q_hbm, o_hbm)

    return run


def kernel(X, X_original):
    return _build(X.shape[0])(X, X_original)


# speedup vs baseline: 1.8316x; 1.8316x over previous
import functools

import jax
import jax.numpy as jnp
from jax.experimental import pallas as pl
from jax.experimental.pallas import tpu as pltpu

_INTERIM = 256
_M = 128
_C_IN = 2 * _INTERIM + 4
_C_OUT = 2 * _M + 4
_SCALE = 255.0 / 1.4
_BLK = 512


def _tc_body(x_ref, q_ref, o_ref):
    x = q_ref[...]
    t = x * _SCALE
    t = jnp.minimum(jnp.maximum(t, 0.0), float(_INTERIM - 1))
    lo = jnp.minimum(t.astype(jnp.int32), _INTERIM - 2)
    frac = t - lo.astype(jnp.float32)

    def interp(tab0, tab1, idx):
        i0 = jnp.minimum(idx, _M - 1)
        i1 = jnp.maximum(idx - _M, 0)
        g0 = jnp.take_along_axis(tab0, i0, axis=-1, mode="promise_in_bounds")
        g1 = jnp.take_along_axis(tab1, i1, axis=-1, mode="promise_in_bounds")
        return jnp.where(idx < _M, g0, g1)

    p0 = x_ref[:, 0:_M]
    p1 = x_ref[:, _M:2 * _M]
    t0 = x_ref[:, 2 * _M:3 * _M]
    t1 = x_ref[:, 3 * _M:4 * _M]
    p_lo = interp(p0, p1, lo)
    p_hi = interp(p0, p1, lo + 1)
    q_lo = interp(t0, t1, lo)
    q_hi = interp(t0, t1, lo + 1)
    o_ref[:, 0:_M] = p_lo + frac * (p_hi - p_lo)
    o_ref[:, _M:2 * _M] = q_lo + frac * (q_hi - q_lo)
    o_ref[:, 2 * _M:_C_OUT] = x_ref[:, 2 * _INTERIM:_C_IN]


@functools.cache
def _build_tc(batch):
    return pl.pallas_call(
        _tc_body,
        grid=(batch // _BLK,),
        in_specs=[
            pl.BlockSpec((_BLK, _C_IN), lambda i: (i, 0)),
            pl.BlockSpec((_BLK, _M), lambda i: (i, 0)),
        ],
        out_specs=pl.BlockSpec((_BLK, _C_OUT), lambda i: (i, 0)),
        out_shape=jax.ShapeDtypeStruct((batch, _C_OUT), jnp.float32),
    )


def kernel(X, X_original):
    return _build_tc(X.shape[0])(X, X_original)
